# baseline (speedup 1.0000x reference)
import jax
import jax.numpy as jnp
from jax import lax
from jax.experimental import pallas as pl
from jax.experimental.pallas import tpu as pltpu
from jax.experimental.pallas import tpu_sc as plsc

N = 10000
E = 320000
NC, NS, L = 2, 16, 16
NW = NC * NS
G = 128
KW = 80
E_PAD = NW * KW * G
NPAD = 10240
RPS = NPAD // NS
TRASH = N
KW0, KW1 = 80, 80
KWMAX = max(KW0, KW1)
G_EXTRA = KWMAX + 8

_f32 = jnp.float32
_bf16 = jnp.bfloat16

import numpy as _np
_SIGMA = _np.concatenate([_np.arange(0, 32, 2), _np.arange(1, 32, 2),
                          32 + _np.arange(0, 32, 2), 32 + _np.arange(1, 32, 2)])
_INV = _np.empty(64, _np.int32)
_INV[_SIGMA] = _np.arange(64)


def _make_sc_agg(D, with_deg=False):
  mesh = plsc.VectorSubcoreMesh(core_axis_name="c", subcore_axis_name="s")
  out_type = [jax.ShapeDtypeStruct((NC, NPAD, D), _f32)]
  scratch = [
      pltpu.VMEM((KWMAX + 1, G), jnp.int32),
      pltpu.VMEM((KWMAX, G), jnp.int32),
      pltpu.VMEM((2, G, D), _bf16),
      pltpu.VMEM((G, D), _f32),
      pltpu.VMEM_SHARED((NPAD, D), _f32),
      pltpu.SemaphoreType.DMA((2,)),
  ]
  if with_deg:
    out_type.append(jax.ShapeDtypeStruct((NC, NPAD, L), _f32))
    scratch += [
        pltpu.VMEM((G, L), _f32),
        pltpu.VMEM((G, L), _f32),
        pltpu.VMEM_SHARED((NPAD, L), _f32),
    ]

  def body(edge_hbm, p_hbm, part_hbm, *refs):
    if with_deg:
      (degp_hbm, src_v, dst_v, rows_b, rows_v, acc_sh, sem,
       ones_v, z16_v, deg_sh) = refs
    else:
      src_v, dst_v, rows_b, rows_v, acc_sh, sem = refs
    c = lax.axis_index("c")
    s = lax.axis_index("s")
    kw_c = KW0 + c * (KW1 - KW0)
    base = c * (NS * KW0) + s * kw_c

    pltpu.sync_copy(edge_hbm.at[0, pl.ds(base, KWMAX + 1)], src_v)
    pltpu.sync_copy(edge_hbm.at[1, pl.ds(base, KWMAX)], dst_v)

    def zrow(i, _):
      def zcol(j, _):
        rows_v[i, pl.ds(j * L, L)] = jnp.zeros((L,), _f32)
        return 0
      lax.fori_loop(0, D // L, zcol, 0)
      return 0
    lax.fori_loop(0, G, zrow, 0)

    def zacc(i, _):
      pltpu.sync_copy(rows_v, acc_sh.at[pl.ds(s * RPS + i * G, G)])
      return 0
    lax.fori_loop(0, RPS // G, zacc, 0)

    if with_deg:
      def fill16(i, _):
        ones_v[i, :] = jnp.ones((L,), _f32)
        z16_v[i, :] = jnp.zeros((L,), _f32)
        return 0
      lax.fori_loop(0, G, fill16, 0)

      def zdeg(i, _):
        pltpu.sync_copy(z16_v, deg_sh.at[pl.ds(s * RPS + i * G, G)])
        return 0
      lax.fori_loop(0, RPS // G, zdeg, 0)

    plsc.subcore_barrier()

    pltpu.async_copy(p_hbm.at[src_v.at[0]], rows_b.at[0], sem.at[0])

    def step(jo, _):
      for b in range(2):
        j = jo * 2 + b
        pltpu.async_copy(p_hbm.at[src_v.at[j + 1]], rows_b.at[1 - b],
                         sem.at[1 - b])
        pltpu.make_async_copy(p_hbm.at[pl.ds(0, G)], rows_b.at[b],
                              sem.at[b]).wait()

        def conv(i2, _):
          for r in range(2):
            i = i2 * 2 + r
            for h in range(D // 32):
              v = rows_b[b, i, pl.ds(32 * h, 32)]
              ev, od = plsc.unpack(v, format=plsc.PackFormat.INTERLEAVED)
              rows_v[i, pl.ds(32 * h, L)] = ev
              rows_v[i, pl.ds(32 * h + L, L)] = od
          return 0
        lax.fori_loop(0, G // 2, conv, 0)

        pltpu.sync_copy(rows_v, acc_sh.at[dst_v.at[j]], add=True)
        if with_deg:
          pltpu.sync_copy(ones_v, deg_sh.at[dst_v.at[j]], add=True)
      return 0
    lax.fori_loop(0, kw_c // 2, step, 0)

    pltpu.make_async_copy(p_hbm.at[pl.ds(0, G)], rows_b.at[0],
                          sem.at[0]).wait()

    plsc.subcore_barrier()

    def wout(i, _):
      off = s * RPS + i * G
      pltpu.sync_copy(acc_sh.at[pl.ds(off, G)],
                      part_hbm.at[c, pl.ds(off, G)])
      if with_deg:
        pltpu.sync_copy(deg_sh.at[pl.ds(off, G)],
                        degp_hbm.at[c, pl.ds(off, G)])
      return 0
    lax.fori_loop(0, RPS // G, wout, 0)

  return pl.kernel(body, out_type=tuple(out_type), mesh=mesh,
                   scratch_types=scratch,
                   compiler_params=pltpu.CompilerParams(
                       use_tc_tiling_on_sc=False,
                       needs_layout_passes=False))


_agg64 = _make_sc_agg(64)
_agg64_deg = _make_sc_agg(64, with_deg=True)


_BM = 2000


def _mm_body(x_ref, w_ref, o1_ref, o2_ref):
  p = jnp.dot(x_ref[...], w_ref[...], preferred_element_type=_f32)
  o1_ref[...] = p[:, :64].astype(_bf16)
  o2_ref[...] = p[:, 64:].astype(_bf16)


def _matmul_p1(x, w):
  return pl.pallas_call(
      _mm_body,
      grid=(N // _BM,),
      in_specs=[pl.BlockSpec((_BM, 128), lambda i: (i, 0)),
                pl.BlockSpec((128, 128), lambda i: (0, 0))],
      out_specs=[pl.BlockSpec((_BM, 64), lambda i: (i, 0)),
                 pl.BlockSpec((_BM, 64), lambda i: (i, 0))],
      out_shape=[jax.ShapeDtypeStruct((N, 64), _bf16),
                 jax.ShapeDtypeStruct((N, 64), _bf16)],
  )(x, w)


def _mid_body(sa_ref, sb_ref, degp_ref, x_ref, w1r_ref, b1_ref, w2l_ref,
              w2r_ref, b2_ref, p2_ref, r2_ref):
  deg = degp_ref[0, :, 0] + degp_ref[1, :, 0]
  dinv = 1.0 / jnp.maximum(deg, 1.0)
  agg = jnp.concatenate([sa_ref[0] + sa_ref[1], sb_ref[0] + sb_ref[1]],
                        axis=1) * dinv[:, None]
  h = agg + jnp.dot(x_ref[...], w1r_ref[...], preferred_element_type=_f32)
  h = jnp.maximum(h + b1_ref[...], 0.0)
  p2_ref[...] = jnp.dot(h, w2l_ref[...],
                        preferred_element_type=_f32).astype(_bf16)
  r2_ref[...] = (jnp.dot(h, w2r_ref[...], preferred_element_type=_f32)
                 + b2_ref[...])


def _mid(Sa, Sb, degp, x, w1r, b1, w2l, w2r, b2):
  return pl.pallas_call(
      _mid_body,
      grid=(N // _BM,),
      in_specs=[
          pl.BlockSpec((NC, _BM, 64), lambda i: (0, i, 0)),
          pl.BlockSpec((NC, _BM, 64), lambda i: (0, i, 0)),
          pl.BlockSpec((NC, _BM, L), lambda i: (0, i, 0)),
          pl.BlockSpec((_BM, 128), lambda i: (i, 0)),
          pl.BlockSpec((128, 128), lambda i: (0, 0)),
          pl.BlockSpec((1, 128), lambda i: (0, 0)),
          pl.BlockSpec((128, 64), lambda i: (0, 0)),
          pl.BlockSpec((128, 64), lambda i: (0, 0)),
          pl.BlockSpec((1, 64), lambda i: (0, 0)),
      ],
      out_specs=[pl.BlockSpec((_BM, 64), lambda i: (i, 0)),
                 pl.BlockSpec((_BM, 64), lambda i: (i, 0))],
      out_shape=[jax.ShapeDtypeStruct((N, 64), _bf16),
                 jax.ShapeDtypeStruct((N, 64), _f32)],
  )(Sa, Sb, degp, x, w1r, b1, w2l, w2r, b2)


def _final_body(t_ref, degp_ref, r2_ref, o_ref):
  deg = degp_ref[0, :, 0] + degp_ref[1, :, 0]
  dinv = 1.0 / jnp.maximum(deg, 1.0)
  o_ref[...] = (t_ref[0] + t_ref[1]) * dinv[:, None] + r2_ref[...]


def _final(T, degp, r2):
  return pl.pallas_call(
      _final_body,
      grid=(N // _BM,),
      in_specs=[
          pl.BlockSpec((NC, _BM, 64), lambda i: (0, i, 0)),
          pl.BlockSpec((NC, _BM, L), lambda i: (0, i, 0)),
          pl.BlockSpec((_BM, 64), lambda i: (i, 0)),
      ],
      out_specs=pl.BlockSpec((_BM, 64), lambda i: (i, 0)),
      out_shape=jax.ShapeDtypeStruct((N, 64), _f32),
  )(T, degp, r2)


def kernel(x, edge_index, W1_l, b1, W1_r, W2_l, b2, W2_r):
  src = edge_index[0].astype(jnp.int32)
  dst = edge_index[1].astype(jnp.int32)
  pad = E_PAD - E + G_EXTRA * G
  srcp = jnp.concatenate([src, jnp.zeros((pad,), jnp.int32)])
  trash = TRASH + jnp.arange(pad, dtype=jnp.int32) % (NPAD - N)
  dstp = jnp.concatenate([dst, trash])
  edges = jnp.stack([srcp, dstp]).reshape(2, NW * KW + G_EXTRA, G)

  w1lp = W1_l[:, _np.concatenate([_INV, 64 + _INV])]
  w2lp = W2_l[:, _INV]

  p1a, p1b = _matmul_p1(x, w1lp)
  Sa, degp = _agg64_deg(edges, p1a)
  (Sb,) = _agg64(edges, p1b)
  p2, r2 = _mid(Sa, Sb, degp, x, W1_r, b1.reshape(1, -1), w2lp, W2_r,
                b2.reshape(1, -1))
  (T,) = _agg64(edges, p2)
  return _final(T, degp, r2)


# speedup vs baseline: 1.0148x; 1.0148x over previous
import jax
import jax.numpy as jnp
from jax import lax
from jax.experimental import pallas as pl
from jax.experimental.pallas import tpu as pltpu
from jax.experimental.pallas import tpu_sc as plsc

N = 10000
E = 320000
NC, NS, L = 2, 16, 16
NW = NC * NS
G = 128
KW = 80
E_PAD = NW * KW * G
NPAD = 10240
RPS = NPAD // NS
TRASH = N
KW0, KW1 = 88, 72
KWMAX = max(KW0, KW1)
G_EXTRA = KWMAX + 8

_f32 = jnp.float32
_bf16 = jnp.bfloat16

import numpy as _np
_SIGMA = _np.concatenate([_np.arange(0, 32, 2), _np.arange(1, 32, 2),
                          32 + _np.arange(0, 32, 2), 32 + _np.arange(1, 32, 2)])
_INV = _np.empty(64, _np.int32)
_INV[_SIGMA] = _np.arange(64)


def _make_sc_agg(D, with_deg=False):
  mesh = plsc.VectorSubcoreMesh(core_axis_name="c", subcore_axis_name="s")
  out_type = [jax.ShapeDtypeStruct((NC, NPAD, D), _f32)]
  scratch = [
      pltpu.VMEM((KWMAX + 1, G), jnp.int32),
      pltpu.VMEM((KWMAX, G), jnp.int32),
      pltpu.VMEM((2, G, D), _bf16),
      pltpu.VMEM((G, D), _f32),
      pltpu.VMEM_SHARED((NPAD, D), _f32),
      pltpu.SemaphoreType.DMA((2,)),
  ]
  if with_deg:
    out_type.append(jax.ShapeDtypeStruct((NC, NPAD, L), _f32))
    scratch += [
        pltpu.VMEM((G, L), _f32),
        pltpu.VMEM((G, L), _f32),
        pltpu.VMEM_SHARED((NPAD, L), _f32),
    ]

  def body(edge_hbm, p_hbm, part_hbm, *refs):
    if with_deg:
      (degp_hbm, src_v, dst_v, rows_b, rows_v, acc_sh, sem,
       ones_v, z16_v, deg_sh) = refs
    else:
      src_v, dst_v, rows_b, rows_v, acc_sh, sem = refs
    c = lax.axis_index("c")
    s = lax.axis_index("s")
    kw_c = KW0 + c * (KW1 - KW0)
    base = c * (NS * KW0) + s * kw_c

    pltpu.sync_copy(edge_hbm.at[0, pl.ds(base, KWMAX + 1)], src_v)
    pltpu.sync_copy(edge_hbm.at[1, pl.ds(base, KWMAX)], dst_v)

    def zrow(i, _):
      def zcol(j, _):
        rows_v[i, pl.ds(j * L, L)] = jnp.zeros((L,), _f32)
        return 0
      lax.fori_loop(0, D // L, zcol, 0)
      return 0
    lax.fori_loop(0, G, zrow, 0)

    def zacc(i, _):
      pltpu.sync_copy(rows_v, acc_sh.at[pl.ds(s * RPS + i * G, G)])
      return 0
    lax.fori_loop(0, RPS // G, zacc, 0)

    if with_deg:
      def fill16(i, _):
        ones_v[i, :] = jnp.ones((L,), _f32)
        z16_v[i, :] = jnp.zeros((L,), _f32)
        return 0
      lax.fori_loop(0, G, fill16, 0)

      def zdeg(i, _):
        pltpu.sync_copy(z16_v, deg_sh.at[pl.ds(s * RPS + i * G, G)])
        return 0
      lax.fori_loop(0, RPS // G, zdeg, 0)

    plsc.subcore_barrier()

    pltpu.async_copy(p_hbm.at[src_v.at[0]], rows_b.at[0], sem.at[0])

    def step(jo, _):
      for b in range(2):
        j = jo * 2 + b
        pltpu.async_copy(p_hbm.at[src_v.at[j + 1]], rows_b.at[1 - b],
                         sem.at[1 - b])
        pltpu.make_async_copy(p_hbm.at[pl.ds(0, G)], rows_b.at[b],
                              sem.at[b]).wait()

        def conv(i2, _):
          for r in range(2):
            i = i2 * 2 + r
            for h in range(D // 32):
              v = rows_b[b, i, pl.ds(32 * h, 32)]
              ev, od = plsc.unpack(v, format=plsc.PackFormat.INTERLEAVED)
              rows_v[i, pl.ds(32 * h, L)] = ev
              rows_v[i, pl.ds(32 * h + L, L)] = od
          return 0
        lax.fori_loop(0, G // 2, conv, 0)

        pltpu.sync_copy(rows_v, acc_sh.at[dst_v.at[j]], add=True)
        if with_deg:
          pltpu.sync_copy(ones_v, deg_sh.at[dst_v.at[j]], add=True)
      return 0
    lax.fori_loop(0, kw_c // 2, step, 0)

    pltpu.make_async_copy(p_hbm.at[pl.ds(0, G)], rows_b.at[0],
                          sem.at[0]).wait()

    plsc.subcore_barrier()

    def wout(i, _):
      off = s * RPS + i * G
      pltpu.sync_copy(acc_sh.at[pl.ds(off, G)],
                      part_hbm.at[c, pl.ds(off, G)])
      if with_deg:
        pltpu.sync_copy(deg_sh.at[pl.ds(off, G)],
                        degp_hbm.at[c, pl.ds(off, G)])
      return 0
    lax.fori_loop(0, RPS // G, wout, 0)

  return pl.kernel(body, out_type=tuple(out_type), mesh=mesh,
                   scratch_types=scratch,
                   compiler_params=pltpu.CompilerParams(
                       use_tc_tiling_on_sc=False,
                       needs_layout_passes=False))


_agg64 = _make_sc_agg(64)
_agg64_deg = _make_sc_agg(64, with_deg=True)


_BM = 2000


def _mm_body(x_ref, w_ref, o1_ref, o2_ref):
  p = jnp.dot(x_ref[...], w_ref[...], preferred_element_type=_f32)
  o1_ref[...] = p[:, :64].astype(_bf16)
  o2_ref[...] = p[:, 64:].astype(_bf16)


def _matmul_p1(x, w):
  return pl.pallas_call(
      _mm_body,
      grid=(N // _BM,),
      in_specs=[pl.BlockSpec((_BM, 128), lambda i: (i, 0)),
                pl.BlockSpec((128, 128), lambda i: (0, 0))],
      out_specs=[pl.BlockSpec((_BM, 64), lambda i: (i, 0)),
                 pl.BlockSpec((_BM, 64), lambda i: (i, 0))],
      out_shape=[jax.ShapeDtypeStruct((N, 64), _bf16),
                 jax.ShapeDtypeStruct((N, 64), _bf16)],
  )(x, w)


def _mid_body(sa_ref, sb_ref, degp_ref, x_ref, w1r_ref, b1_ref, w2l_ref,
              w2r_ref, b2_ref, p2_ref, r2_ref):
  deg = degp_ref[0, :, 0] + degp_ref[1, :, 0]
  dinv = 1.0 / jnp.maximum(deg, 1.0)
  agg = jnp.concatenate([sa_ref[0] + sa_ref[1], sb_ref[0] + sb_ref[1]],
                        axis=1) * dinv[:, None]
  h = agg + jnp.dot(x_ref[...], w1r_ref[...], preferred_element_type=_f32)
  h = jnp.maximum(h + b1_ref[...], 0.0)
  p2_ref[...] = jnp.dot(h, w2l_ref[...],
                        preferred_element_type=_f32).astype(_bf16)
  r2_ref[...] = (jnp.dot(h, w2r_ref[...], preferred_element_type=_f32)
                 + b2_ref[...])


def _mid(Sa, Sb, degp, x, w1r, b1, w2l, w2r, b2):
  return pl.pallas_call(
      _mid_body,
      grid=(N // _BM,),
      in_specs=[
          pl.BlockSpec((NC, _BM, 64), lambda i: (0, i, 0)),
          pl.BlockSpec((NC, _BM, 64), lambda i: (0, i, 0)),
          pl.BlockSpec((NC, _BM, L), lambda i: (0, i, 0)),
          pl.BlockSpec((_BM, 128), lambda i: (i, 0)),
          pl.BlockSpec((128, 128), lambda i: (0, 0)),
          pl.BlockSpec((1, 128), lambda i: (0, 0)),
          pl.BlockSpec((128, 64), lambda i: (0, 0)),
          pl.BlockSpec((128, 64), lambda i: (0, 0)),
          pl.BlockSpec((1, 64), lambda i: (0, 0)),
      ],
      out_specs=[pl.BlockSpec((_BM, 64), lambda i: (i, 0)),
                 pl.BlockSpec((_BM, 64), lambda i: (i, 0))],
      out_shape=[jax.ShapeDtypeStruct((N, 64), _bf16),
                 jax.ShapeDtypeStruct((N, 64), _f32)],
  )(Sa, Sb, degp, x, w1r, b1, w2l, w2r, b2)


def _final_body(t_ref, degp_ref, r2_ref, o_ref):
  deg = degp_ref[0, :, 0] + degp_ref[1, :, 0]
  dinv = 1.0 / jnp.maximum(deg, 1.0)
  o_ref[...] = (t_ref[0] + t_ref[1]) * dinv[:, None] + r2_ref[...]


def _final(T, degp, r2):
  return pl.pallas_call(
      _final_body,
      grid=(N // _BM,),
      in_specs=[
          pl.BlockSpec((NC, _BM, 64), lambda i: (0, i, 0)),
          pl.BlockSpec((NC, _BM, L), lambda i: (0, i, 0)),
          pl.BlockSpec((_BM, 64), lambda i: (i, 0)),
      ],
      out_specs=pl.BlockSpec((_BM, 64), lambda i: (i, 0)),
      out_shape=jax.ShapeDtypeStruct((N, 64), _f32),
  )(T, degp, r2)


def kernel(x, edge_index, W1_l, b1, W1_r, W2_l, b2, W2_r):
  src = edge_index[0].astype(jnp.int32)
  dst = edge_index[1].astype(jnp.int32)
  pad = E_PAD - E + G_EXTRA * G
  srcp = jnp.concatenate([src, jnp.zeros((pad,), jnp.int32)])
  trash = TRASH + jnp.arange(pad, dtype=jnp.int32) % (NPAD - N)
  dstp = jnp.concatenate([dst, trash])
  edges = jnp.stack([srcp, dstp]).reshape(2, NW * KW + G_EXTRA, G)

  w1lp = W1_l[:, _np.concatenate([_INV, 64 + _INV])]
  w2lp = W2_l[:, _INV]

  p1a, p1b = _matmul_p1(x, w1lp)
  Sa, degp = _agg64_deg(edges, p1a)
  (Sb,) = _agg64(edges, p1b)
  p2, r2 = _mid(Sa, Sb, degp, x, W1_r, b1.reshape(1, -1), w2lp, W2_r,
                b2.reshape(1, -1))
  (T,) = _agg64(edges, p2)
  return _final(T, degp, r2)
